# baseline (speedup 1.0000x reference)
import functools
import jax
import jax.numpy as jnp
from jax import lax
from jax.experimental import pallas as pl
from jax.experimental.pallas import tpu as pltpu
from jax.experimental.pallas import tpu_sc as plsc

_N = 10000
_E = 320000
_D = 128

_NC = 2
_NS = 16
_NW = _NC * _NS
_CHUNK = 64
_EPW = 10240
_EPAD = _NW * _EPW
_NCHUNK = _EPW // _CHUNK
_TOTCH = _EPAD // _CHUNK
_K0 = 160
_K1 = 160
assert _NS * (_K0 + _K1) == _TOTCH
_NPAD = 10240
_RPT = _NPAD // _NS
_DEGW = 16

_R = 2000

_mesh = plsc.VectorSubcoreMesh(core_axis_name="c", subcore_axis_name="s")


def _deg_body(dst_hbm, ones_hbm, zero_hbm, out_hbm, dsts_v, ones_v, acc_sh, sem):
    c = lax.axis_index("c")
    s = lax.axis_index("s")
    wid = c * _NS + s
    pltpu.sync_copy(zero_hbm, acc_sh.at[pl.ds(s * _RPT, _RPT)])
    pltpu.sync_copy(ones_hbm, ones_v)
    pltpu.sync_copy(dst_hbm.at[wid], dsts_v)
    plsc.subcore_barrier()

    def body(j, carry):
        pltpu.sync_copy(ones_v, acc_sh.at[dsts_v.at[j]], add=True)
        return carry

    lax.fori_loop(0, _NCHUNK, body, 0)
    plsc.subcore_barrier()
    pltpu.sync_copy(acc_sh.at[pl.ds(s * _RPT, _RPT)],
                    out_hbm.at[c, pl.ds(s * _RPT, _RPT)])


_deg_call = functools.partial(
    pl.kernel,
    out_type=jax.ShapeDtypeStruct((_NC, _NPAD, _DEGW), jnp.float32),
    mesh=_mesh,
    scratch_types=[
        pltpu.VMEM((_NCHUNK, _CHUNK), jnp.int32),
        pltpu.VMEM((_CHUNK, _DEGW), jnp.float32),
        pltpu.VMEM_SHARED((_NPAD, _DEGW), jnp.float32),
        pltpu.SemaphoreType.DMA,
    ],
    compiler_params=pltpu.CompilerParams(use_tc_tiling_on_sc=False),
)(_deg_body)


def _agg_body(zp_hbm, sd_hbm, zero_hbm, out_hbm,
              sd_a, sd_b, sd_c, sd_d, rows_a, rows_b, rows_c, rows_d, acc_sh,
              iA, iB, iC, iD, gA, gB, gC, gD, sA, sB, sC, sD):
    c = lax.axis_index("c")
    s = lax.axis_index("s")
    base = jnp.where(c == 0, s * _K0, _NS * _K0 + s * _K1)
    cnt = jnp.where(c == 0, _K0, _K1)
    pltpu.sync_copy(zero_hbm, acc_sh.at[pl.ds(s * _RPT, _RPT)])
    plsc.subcore_barrier()

    sds = (sd_a, sd_b, sd_c, sd_d)
    rows = (rows_a, rows_b, rows_c, rows_d)
    gsem = (gA, gB, gC, gD)
    isem = (iA, iB, iC, iD)
    ssem = (sA, sB, sC, sD)

    def idx_load(k, j):
        return pltpu.async_copy(sd_hbm.at[j], sds[k], isem[k])

    def idx_wait(k):
        pltpu.make_async_copy(sd_hbm.at[base], sds[k], isem[k]).wait()

    def gather(k):
        return pltpu.async_copy(zp_hbm.at[sds[k].at[0]], rows[k], gsem[k])

    def gather_wait(k):
        pltpu.make_async_copy(zp_hbm.at[sds[k].at[0]], rows[k],
                              gsem[k]).wait()

    def scat(k):
        return pltpu.async_copy(rows[k], acc_sh.at[sds[k].at[1]], ssem[k],
                                add=True)

    def scat_wait(k):
        pltpu.make_async_copy(rows[k], acc_sh.at[pl.ds(0, _CHUNK)],
                              ssem[k]).wait()

    for k in range(4):
        idx_load(k, base + k)
    for k in range(3):
        idx_wait(k)
        gather(k)

    def body(i, carry):
        j = 4 * i

        def nxt(k):
            return base + jnp.minimum(j + 4 + k, cnt - 1)

        gather_wait(0); scat(0)
        idx_wait(3);    gather(3)
        gather_wait(1); scat(1)
        scat_wait(0);   idx_load(0, nxt(0))
        gather_wait(2); scat(2)
        scat_wait(1);   idx_load(1, nxt(1))
        gather_wait(3); scat(3)
        scat_wait(2);   idx_load(2, nxt(2))
        idx_wait(0);    gather(0)
        scat_wait(3);   idx_load(3, nxt(3))
        idx_wait(1);    gather(1)
        idx_wait(2);    gather(2)
        return carry

    lax.fori_loop(0, cnt // 4, body, 0)
    gather_wait(0)
    gather_wait(1)
    gather_wait(2)
    idx_wait(3)
    plsc.subcore_barrier()
    pltpu.sync_copy(acc_sh.at[pl.ds(s * _RPT, _RPT)],
                    out_hbm.at[c, pl.ds(s * _RPT, _RPT)])


_agg_call = functools.partial(
    pl.kernel,
    out_type=jax.ShapeDtypeStruct((_NC, _NPAD, _D), jnp.float32),
    mesh=_mesh,
    scratch_types=(
        [pltpu.VMEM((2, _CHUNK), jnp.int32)] * 4
        + [pltpu.VMEM((_CHUNK, _D), jnp.float32)] * 4
        + [pltpu.VMEM_SHARED((_NPAD, _D), jnp.float32)]
        + [pltpu.SemaphoreType.DMA] * 12
    ),
)(_agg_body)


def _b0_body(deg_ref, x_ref, w_ref, dinv_ref, zp_ref):
    deg = jnp.sum(deg_ref[...], axis=(0, 2)) + 1.0
    dinv = lax.rsqrt(deg)
    dinv_ref[...] = dinv[:, None]
    zp_ref[...] = dinv[:, None] * jnp.dot(
        x_ref[...], w_ref[...], preferred_element_type=jnp.float32)


_b0_call = pl.pallas_call(
    _b0_body,
    grid=(_N // _R,),
    in_specs=[
        pl.BlockSpec((_NC, _R, _DEGW), lambda i: (0, i, 0)),
        pl.BlockSpec((_R, _D), lambda i: (i, 0)),
        pl.BlockSpec((_D, _D), lambda i: (0, 0)),
    ],
    out_specs=[
        pl.BlockSpec((_R, 1), lambda i: (i, 0)),
        pl.BlockSpec((_R, _D), lambda i: (i, 0)),
    ],
    out_shape=[
        jax.ShapeDtypeStruct((_N, 1), jnp.float32),
        jax.ShapeDtypeStruct((_N, _D), jnp.float32),
    ],
)


def _mid_body(parts_ref, zp_ref, dinv_ref, b_ref, w_ref, out_ref):
    dinv = dinv_ref[...]
    y = dinv * (parts_ref[0] + parts_ref[1] + zp_ref[...]) + b_ref[...]
    out_ref[...] = dinv * jnp.dot(
        y, w_ref[...], preferred_element_type=jnp.float32)


_mid_call = pl.pallas_call(
    _mid_body,
    grid=(_N // _R,),
    in_specs=[
        pl.BlockSpec((_NC, _R, _D), lambda i: (0, i, 0)),
        pl.BlockSpec((_R, _D), lambda i: (i, 0)),
        pl.BlockSpec((_R, 1), lambda i: (i, 0)),
        pl.BlockSpec((1, _D), lambda i: (0, 0)),
        pl.BlockSpec((_D, _D), lambda i: (0, 0)),
    ],
    out_specs=pl.BlockSpec((_R, _D), lambda i: (i, 0)),
    out_shape=jax.ShapeDtypeStruct((_N, _D), jnp.float32),
)


def _fin_body(parts_ref, zp_ref, dinv_ref, b_ref, out_ref):
    out_ref[...] = dinv_ref[...] * (
        parts_ref[0] + parts_ref[1] + zp_ref[...]) + b_ref[...]


_fin_call = pl.pallas_call(
    _fin_body,
    grid=(_N // _R,),
    in_specs=[
        pl.BlockSpec((_NC, _R, _D), lambda i: (0, i, 0)),
        pl.BlockSpec((_R, _D), lambda i: (i, 0)),
        pl.BlockSpec((_R, 1), lambda i: (i, 0)),
        pl.BlockSpec((1, _D), lambda i: (0, 0)),
    ],
    out_specs=pl.BlockSpec((_R, _D), lambda i: (i, 0)),
    out_shape=jax.ShapeDtypeStruct((_N, _D), jnp.float32),
)


def kernel(x, edge_index, W1, b1, W2, b2, W3, b3):
    pad = _EPAD - _E
    src = jnp.concatenate(
        [edge_index[0].astype(jnp.int32), jnp.zeros((pad,), jnp.int32)]
    ).reshape(_NW, _NCHUNK, _CHUNK)
    trash = _N + (jnp.arange(pad, dtype=jnp.int32) % (_NPAD - _N))
    dst = jnp.concatenate(
        [edge_index[1].astype(jnp.int32), trash]
    ).reshape(_NW, _NCHUNK, _CHUNK)
    sd = jnp.stack([src, dst], axis=2).reshape(_TOTCH, 2, _CHUNK)

    ones_deg = jnp.full((_CHUNK, _DEGW), 1.0 / _DEGW, jnp.float32)
    zero_deg = jnp.zeros((_RPT, _DEGW), jnp.float32)
    zero_acc = jnp.zeros((_RPT, _D), jnp.float32)

    deg_parts = _deg_call(dst, ones_deg, zero_deg)
    dinv, z1 = _b0_call(deg_parts, x, W1)
    p1 = _agg_call(z1, sd, zero_acc)
    z2 = _mid_call(p1, z1, dinv, b1.reshape(1, _D), W2)
    p2 = _agg_call(z2, sd, zero_acc)
    z3 = _mid_call(p2, z2, dinv, b2.reshape(1, _D), W3)
    p3 = _agg_call(z3, sd, zero_acc)
    return _fin_call(p3, z3, dinv, b3.reshape(1, _D))


# speedup vs baseline: 2.2930x; 2.2930x over previous
import functools
import jax
import jax.numpy as jnp
from jax import lax
from jax.experimental import pallas as pl
from jax.experimental.pallas import tpu as pltpu
from jax.experimental.pallas import tpu_sc as plsc

_N = 10000
_E = 320000
_D = 128
_H = _D // 2

_NC = 2
_NS = 16
_NW = _NC * _NS
_CHUNK = 128
_EPW = 10240
_EPAD = _NW * _EPW
_NCHUNK = _EPW // _CHUNK
_TOTCH = _EPAD // _CHUNK
_CPS = _TOTCH // _NS
_NPAD = 10240
_RPT = _NPAD // _NS
_SRT = _N // _NS
_DEGW = 16

_R = 2000

_mesh = plsc.VectorSubcoreMesh(core_axis_name="c", subcore_axis_name="s")
_untiled = pltpu.CompilerParams(use_tc_tiling_on_sc=False)


def _deg_body(dst_hbm, ones_hbm, zero_hbm, out_hbm, dsts_v, ones_v, acc_sh, sem):
    c = lax.axis_index("c")
    s = lax.axis_index("s")
    wid = c * _NS + s
    pltpu.sync_copy(zero_hbm, acc_sh.at[pl.ds(s * _RPT, _RPT)])
    pltpu.sync_copy(ones_hbm, ones_v)
    pltpu.sync_copy(dst_hbm.at[wid], dsts_v)
    plsc.subcore_barrier()

    def body(j, carry):
        pltpu.sync_copy(ones_v, acc_sh.at[dsts_v.at[j]], add=True)
        return carry

    lax.fori_loop(0, _NCHUNK, body, 0)
    plsc.subcore_barrier()
    pltpu.sync_copy(acc_sh.at[pl.ds(s * _RPT, _RPT)],
                    out_hbm.at[c, pl.ds(s * _RPT, _RPT)])


_deg_call = functools.partial(
    pl.kernel,
    out_type=jax.ShapeDtypeStruct((_NC, _NPAD, _DEGW), jnp.float32),
    mesh=_mesh,
    scratch_types=[
        pltpu.VMEM((_NCHUNK, _CHUNK), jnp.int32),
        pltpu.VMEM((_CHUNK, _DEGW), jnp.float32),
        pltpu.VMEM_SHARED((_NPAD, _DEGW), jnp.float32),
        pltpu.SemaphoreType.DMA,
    ],
    compiler_params=_untiled,
)(_deg_body)


def _agg_body(zh_hbm, sd_hbm, zero_hbm, out_hbm,
              sd_a, sd_b, sd_c, sd_d, rows_a, rows_b, rows_c, rows_d,
              rep_sh, acc_sh,
              iA, iB, iC, iD, gA, gB, gC, gD, sA, sB, sC, sD):
    c = lax.axis_index("c")
    s = lax.axis_index("s")
    base = s * _CPS
    pltpu.sync_copy(zero_hbm, acc_sh.at[pl.ds(s * _RPT, _RPT)])
    pltpu.sync_copy(zh_hbm.at[c, pl.ds(s * _SRT, _SRT)],
                    rep_sh.at[pl.ds(s * _SRT, _SRT)])
    plsc.subcore_barrier()

    sds = (sd_a, sd_b, sd_c, sd_d)
    rows = (rows_a, rows_b, rows_c, rows_d)
    gsem = (gA, gB, gC, gD)
    isem = (iA, iB, iC, iD)
    ssem = (sA, sB, sC, sD)

    def idx_load(k, j):
        return pltpu.async_copy(sd_hbm.at[j], sds[k], isem[k])

    def idx_wait(k):
        pltpu.make_async_copy(sd_hbm.at[0], sds[k], isem[k]).wait()

    def gather(k):
        return pltpu.async_copy(rep_sh.at[sds[k].at[0]], rows[k], gsem[k])

    def gather_wait(k):
        pltpu.make_async_copy(rep_sh.at[sds[k].at[0]], rows[k],
                              gsem[k]).wait()

    def scat(k):
        return pltpu.async_copy(rows[k], acc_sh.at[sds[k].at[1]], ssem[k],
                                add=True)

    def scat_wait(k):
        pltpu.make_async_copy(rows[k], acc_sh.at[pl.ds(0, _CHUNK)],
                              ssem[k]).wait()

    for k in range(4):
        idx_load(k, base + k)
    for k in range(3):
        idx_wait(k)
        gather(k)

    def body(i, carry):
        j = 4 * i

        def nxt(k):
            return base + jnp.minimum(j + 4 + k, _CPS - 1)

        gather_wait(0); scat(0)
        idx_wait(3);    gather(3)
        gather_wait(1); scat(1)
        scat_wait(0);   idx_load(0, nxt(0))
        gather_wait(2); scat(2)
        scat_wait(1);   idx_load(1, nxt(1))
        gather_wait(3); scat(3)
        scat_wait(2);   idx_load(2, nxt(2))
        idx_wait(0);    gather(0)
        scat_wait(3);   idx_load(3, nxt(3))
        idx_wait(1);    gather(1)
        idx_wait(2);    gather(2)
        return carry

    lax.fori_loop(0, _CPS // 4, body, 0)
    gather_wait(0)
    gather_wait(1)
    gather_wait(2)
    idx_wait(3)
    plsc.subcore_barrier()
    pltpu.sync_copy(acc_sh.at[pl.ds(s * _RPT, _RPT)],
                    out_hbm.at[c, pl.ds(s * _RPT, _RPT)])


_agg_call = functools.partial(
    pl.kernel,
    out_type=jax.ShapeDtypeStruct((_NC, _NPAD, _H), jnp.float32),
    mesh=_mesh,
    scratch_types=(
        [pltpu.VMEM((2, _CHUNK), jnp.int32)] * 4
        + [pltpu.VMEM((_CHUNK, _H), jnp.float32)] * 4
        + [pltpu.VMEM_SHARED((_N, _H), jnp.float32)]
        + [pltpu.VMEM_SHARED((_NPAD, _H), jnp.float32)]
        + [pltpu.SemaphoreType.DMA] * 12
    ),
    compiler_params=_untiled,
)(_agg_body)


def _b0_body(deg_ref, x_ref, w_ref, dinv_ref, zh_ref):
    deg = jnp.sum(deg_ref[...], axis=(0, 2)) + 1.0
    dinv = lax.rsqrt(deg)
    dinv_ref[...] = dinv[:, None]
    z = dinv[:, None] * jnp.dot(
        x_ref[...], w_ref[...], preferred_element_type=jnp.float32)
    zh_ref[0] = z[:, :_H]
    zh_ref[1] = z[:, _H:]


_b0_call = pl.pallas_call(
    _b0_body,
    grid=(_N // _R,),
    in_specs=[
        pl.BlockSpec((_NC, _R, _DEGW), lambda i: (0, i, 0)),
        pl.BlockSpec((_R, _D), lambda i: (i, 0)),
        pl.BlockSpec((_D, _D), lambda i: (0, 0)),
    ],
    out_specs=[
        pl.BlockSpec((_R, 1), lambda i: (i, 0)),
        pl.BlockSpec((_NC, _R, _H), lambda i: (0, i, 0)),
    ],
    out_shape=[
        jax.ShapeDtypeStruct((_N, 1), jnp.float32),
        jax.ShapeDtypeStruct((_NC, _N, _H), jnp.float32),
    ],
)


def _mid_body(parts_ref, zh_ref, dinv_ref, b_ref, w_ref, out_ref):
    dinv = dinv_ref[...]
    acc = jnp.concatenate([parts_ref[0], parts_ref[1]], axis=1)
    zp = jnp.concatenate([zh_ref[0], zh_ref[1]], axis=1)
    y = dinv * (acc + zp) + b_ref[...]
    z = dinv * jnp.dot(y, w_ref[...], preferred_element_type=jnp.float32)
    out_ref[0] = z[:, :_H]
    out_ref[1] = z[:, _H:]


_mid_call = pl.pallas_call(
    _mid_body,
    grid=(_N // _R,),
    in_specs=[
        pl.BlockSpec((_NC, _R, _H), lambda i: (0, i, 0)),
        pl.BlockSpec((_NC, _R, _H), lambda i: (0, i, 0)),
        pl.BlockSpec((_R, 1), lambda i: (i, 0)),
        pl.BlockSpec((1, _D), lambda i: (0, 0)),
        pl.BlockSpec((_D, _D), lambda i: (0, 0)),
    ],
    out_specs=pl.BlockSpec((_NC, _R, _H), lambda i: (0, i, 0)),
    out_shape=jax.ShapeDtypeStruct((_NC, _N, _H), jnp.float32),
)


def _fin_body(parts_ref, zh_ref, dinv_ref, b_ref, out_ref):
    acc = jnp.concatenate([parts_ref[0], parts_ref[1]], axis=1)
    zp = jnp.concatenate([zh_ref[0], zh_ref[1]], axis=1)
    out_ref[...] = dinv_ref[...] * (acc + zp) + b_ref[...]


_fin_call = pl.pallas_call(
    _fin_body,
    grid=(_N // _R,),
    in_specs=[
        pl.BlockSpec((_NC, _R, _H), lambda i: (0, i, 0)),
        pl.BlockSpec((_NC, _R, _H), lambda i: (0, i, 0)),
        pl.BlockSpec((_R, 1), lambda i: (i, 0)),
        pl.BlockSpec((1, _D), lambda i: (0, 0)),
    ],
    out_specs=pl.BlockSpec((_R, _D), lambda i: (i, 0)),
    out_shape=jax.ShapeDtypeStruct((_N, _D), jnp.float32),
)


def kernel(x, edge_index, W1, b1, W2, b2, W3, b3):
    pad = _EPAD - _E
    src = jnp.concatenate(
        [edge_index[0].astype(jnp.int32), jnp.zeros((pad,), jnp.int32)]
    ).reshape(_NW, _NCHUNK, _CHUNK)
    trash = _N + (jnp.arange(pad, dtype=jnp.int32) % (_NPAD - _N))
    dst = jnp.concatenate(
        [edge_index[1].astype(jnp.int32), trash]
    ).reshape(_NW, _NCHUNK, _CHUNK)
    sd = jnp.stack([src, dst], axis=2).reshape(_TOTCH, 2, _CHUNK)

    ones_deg = jnp.full((_CHUNK, _DEGW), 1.0 / _DEGW, jnp.float32)
    zero_deg = jnp.zeros((_RPT, _DEGW), jnp.float32)
    zero_acc = jnp.zeros((_RPT, _H), jnp.float32)

    deg_parts = _deg_call(dst, ones_deg, zero_deg)
    dinv, z1 = _b0_call(deg_parts, x, W1)
    p1 = _agg_call(z1, sd, zero_acc)
    z2 = _mid_call(p1, z1, dinv, b1.reshape(1, _D), W2)
    p2 = _agg_call(z2, sd, zero_acc)
    z3 = _mid_call(p2, z2, dinv, b2.reshape(1, _D), W3)
    p3 = _agg_call(z3, sd, zero_acc)
    return _fin_call(p3, z3, dinv, b3.reshape(1, _D))


# speedup vs baseline: 2.3020x; 1.0039x over previous
import functools
import jax
import jax.numpy as jnp
from jax import lax
from jax.experimental import pallas as pl
from jax.experimental.pallas import tpu as pltpu
from jax.experimental.pallas import tpu_sc as plsc

_N = 10000
_E = 320000
_D = 128
_H = _D // 2

_NC = 2
_NS = 16
_NW = _NC * _NS
_CHUNK = 128
_EPW = 10240
_EPAD = _NW * _EPW
_NCHUNK = _EPW // _CHUNK
_TOTCH = _EPAD // _CHUNK
_CPS = _TOTCH // _NS
_NPAD = 10240
_RPT = _NPAD // _NS
_SRT = _N // _NS
_DEGW = 16

_R = 2000

_mesh = plsc.VectorSubcoreMesh(core_axis_name="c", subcore_axis_name="s")
_untiled = pltpu.CompilerParams(use_tc_tiling_on_sc=False)


def _deg_body(dst_hbm, ones_hbm, zero_hbm, out_hbm, dsts_v, ones_v, acc_sh, sem):
    c = lax.axis_index("c")
    s = lax.axis_index("s")
    wid = c * _NS + s
    pltpu.sync_copy(zero_hbm, acc_sh.at[pl.ds(s * _RPT, _RPT)])
    pltpu.sync_copy(ones_hbm, ones_v)
    pltpu.sync_copy(dst_hbm.at[wid], dsts_v)
    plsc.subcore_barrier()

    def body(j, carry):
        pltpu.sync_copy(ones_v, acc_sh.at[dsts_v.at[j]], add=True)
        return carry

    lax.fori_loop(0, _NCHUNK, body, 0)
    plsc.subcore_barrier()
    pltpu.sync_copy(acc_sh.at[pl.ds(s * _RPT, _RPT)],
                    out_hbm.at[c, pl.ds(s * _RPT, _RPT)])


_deg_call = functools.partial(
    pl.kernel,
    out_type=jax.ShapeDtypeStruct((_NC, _NPAD, _DEGW), jnp.float32),
    mesh=_mesh,
    scratch_types=[
        pltpu.VMEM((_NCHUNK, _CHUNK), jnp.int32),
        pltpu.VMEM((_CHUNK, _DEGW), jnp.float32),
        pltpu.VMEM_SHARED((_NPAD, _DEGW), jnp.float32),
        pltpu.SemaphoreType.DMA,
    ],
    compiler_params=_untiled,
)(_deg_body)


def _agg_body(zh_hbm, src_hbm, dst_hbm, zero_hbm, out_hbm,
              sd_a, sd_b, sd_c, sd_d, rows_a, rows_b, rows_c, rows_d,
              rep_sh, acc_sh,
              iA, iB, iC, iD, gA, gB, gC, gD, sA, sB, sC, sD):
    c = lax.axis_index("c")
    s = lax.axis_index("s")
    base = s * _CPS
    pltpu.sync_copy(zero_hbm, acc_sh.at[pl.ds(s * _RPT, _RPT)])
    pltpu.sync_copy(zh_hbm.at[c, pl.ds(s * _SRT, _SRT)],
                    rep_sh.at[pl.ds(s * _SRT, _SRT)])
    plsc.subcore_barrier()

    sds = (sd_a, sd_b, sd_c, sd_d)
    rows = (rows_a, rows_b, rows_c, rows_d)
    gsem = (gA, gB, gC, gD)
    isem = (iA, iB, iC, iD)
    ssem = (sA, sB, sC, sD)

    def idx_load(k, j):
        pltpu.async_copy(src_hbm.at[pl.ds(j * _CHUNK, _CHUNK)],
                         sds[k].at[0], isem[k])
        pltpu.async_copy(dst_hbm.at[pl.ds(j * _CHUNK, _CHUNK)],
                         sds[k].at[1], isem[k])

    def idx_wait(k):
        pltpu.make_async_copy(src_hbm.at[pl.ds(0, _CHUNK)],
                              sds[k].at[0], isem[k]).wait()
        pltpu.make_async_copy(src_hbm.at[pl.ds(0, _CHUNK)],
                              sds[k].at[1], isem[k]).wait()

    def gather(k):
        return pltpu.async_copy(rep_sh.at[sds[k].at[0]], rows[k], gsem[k])

    def gather_wait(k):
        pltpu.make_async_copy(rep_sh.at[sds[k].at[0]], rows[k],
                              gsem[k]).wait()

    def scat(k):
        return pltpu.async_copy(rows[k], acc_sh.at[sds[k].at[1]], ssem[k],
                                add=True)

    def scat_wait(k):
        pltpu.make_async_copy(rows[k], acc_sh.at[pl.ds(0, _CHUNK)],
                              ssem[k]).wait()

    for k in range(4):
        idx_load(k, base + k)
    for k in range(3):
        idx_wait(k)
        gather(k)

    def body(i, carry):
        j = 4 * i

        def nxt(k):
            return base + jnp.minimum(j + 4 + k, _CPS - 1)

        gather_wait(0); scat(0)
        idx_wait(3);    gather(3)
        gather_wait(1); scat(1)
        scat_wait(0);   idx_load(0, nxt(0))
        gather_wait(2); scat(2)
        scat_wait(1);   idx_load(1, nxt(1))
        gather_wait(3); scat(3)
        scat_wait(2);   idx_load(2, nxt(2))
        idx_wait(0);    gather(0)
        scat_wait(3);   idx_load(3, nxt(3))
        idx_wait(1);    gather(1)
        idx_wait(2);    gather(2)
        return carry

    lax.fori_loop(0, _CPS // 4, body, 0)
    gather_wait(0)
    gather_wait(1)
    gather_wait(2)
    idx_wait(3)
    plsc.subcore_barrier()
    pltpu.sync_copy(acc_sh.at[pl.ds(s * _RPT, _RPT)],
                    out_hbm.at[c, pl.ds(s * _RPT, _RPT)])


_agg_call = functools.partial(
    pl.kernel,
    out_type=jax.ShapeDtypeStruct((_NC, _NPAD, _H), jnp.float32),
    mesh=_mesh,
    scratch_types=(
        [pltpu.VMEM((2, _CHUNK), jnp.int32)] * 4
        + [pltpu.VMEM((_CHUNK, _H), jnp.float32)] * 4
        + [pltpu.VMEM_SHARED((_N, _H), jnp.float32)]
        + [pltpu.VMEM_SHARED((_NPAD, _H), jnp.float32)]
        + [pltpu.SemaphoreType.DMA] * 12
    ),
    compiler_params=_untiled,
)(_agg_body)


def _b0_body(deg_ref, x_ref, w_ref, dinv_ref, zh_ref):
    deg = jnp.sum(deg_ref[...], axis=(0, 2)) + 1.0
    dinv = lax.rsqrt(deg)
    dinv_ref[...] = dinv[:, None]
    z = dinv[:, None] * jnp.dot(
        x_ref[...], w_ref[...], preferred_element_type=jnp.float32)
    zh_ref[0] = z[:, :_H]
    zh_ref[1] = z[:, _H:]


_b0_call = pl.pallas_call(
    _b0_body,
    grid=(_N // _R,),
    in_specs=[
        pl.BlockSpec((_NC, _R, _DEGW), lambda i: (0, i, 0)),
        pl.BlockSpec((_R, _D), lambda i: (i, 0)),
        pl.BlockSpec((_D, _D), lambda i: (0, 0)),
    ],
    out_specs=[
        pl.BlockSpec((_R, 1), lambda i: (i, 0)),
        pl.BlockSpec((_NC, _R, _H), lambda i: (0, i, 0)),
    ],
    out_shape=[
        jax.ShapeDtypeStruct((_N, 1), jnp.float32),
        jax.ShapeDtypeStruct((_NC, _N, _H), jnp.float32),
    ],
)


def _mid_body(parts_ref, zh_ref, dinv_ref, b_ref, w_ref, out_ref):
    dinv = dinv_ref[...]
    acc = jnp.concatenate([parts_ref[0], parts_ref[1]], axis=1)
    zp = jnp.concatenate([zh_ref[0], zh_ref[1]], axis=1)
    y = dinv * (acc + zp) + b_ref[...]
    z = dinv * jnp.dot(y, w_ref[...], preferred_element_type=jnp.float32)
    out_ref[0] = z[:, :_H]
    out_ref[1] = z[:, _H:]


_mid_call = pl.pallas_call(
    _mid_body,
    grid=(_N // _R,),
    in_specs=[
        pl.BlockSpec((_NC, _R, _H), lambda i: (0, i, 0)),
        pl.BlockSpec((_NC, _R, _H), lambda i: (0, i, 0)),
        pl.BlockSpec((_R, 1), lambda i: (i, 0)),
        pl.BlockSpec((1, _D), lambda i: (0, 0)),
        pl.BlockSpec((_D, _D), lambda i: (0, 0)),
    ],
    out_specs=pl.BlockSpec((_NC, _R, _H), lambda i: (0, i, 0)),
    out_shape=jax.ShapeDtypeStruct((_NC, _N, _H), jnp.float32),
)


def _fin_body(parts_ref, zh_ref, dinv_ref, b_ref, out_ref):
    acc = jnp.concatenate([parts_ref[0], parts_ref[1]], axis=1)
    zp = jnp.concatenate([zh_ref[0], zh_ref[1]], axis=1)
    out_ref[...] = dinv_ref[...] * (acc + zp) + b_ref[...]


_fin_call = pl.pallas_call(
    _fin_body,
    grid=(_N // _R,),
    in_specs=[
        pl.BlockSpec((_NC, _R, _H), lambda i: (0, i, 0)),
        pl.BlockSpec((_NC, _R, _H), lambda i: (0, i, 0)),
        pl.BlockSpec((_R, 1), lambda i: (i, 0)),
        pl.BlockSpec((1, _D), lambda i: (0, 0)),
    ],
    out_specs=pl.BlockSpec((_R, _D), lambda i: (i, 0)),
    out_shape=jax.ShapeDtypeStruct((_N, _D), jnp.float32),
)


def kernel(x, edge_index, W1, b1, W2, b2, W3, b3):
    pad = _EPAD - _E
    src = jnp.concatenate(
        [edge_index[0].astype(jnp.int32), jnp.zeros((pad,), jnp.int32)])
    trash = _N + (jnp.arange(pad, dtype=jnp.int32) % (_NPAD - _N))
    dst = jnp.concatenate([edge_index[1].astype(jnp.int32), trash])

    ones_deg = jnp.full((_CHUNK, _DEGW), 1.0 / _DEGW, jnp.float32)
    zero_deg = jnp.zeros((_RPT, _DEGW), jnp.float32)
    zero_acc = jnp.zeros((_RPT, _H), jnp.float32)

    deg_parts = _deg_call(dst.reshape(_NW, _NCHUNK, _CHUNK), ones_deg, zero_deg)
    dinv, z1 = _b0_call(deg_parts, x, W1)
    p1 = _agg_call(z1, src, dst, zero_acc)
    z2 = _mid_call(p1, z1, dinv, b1.reshape(1, _D), W2)
    p2 = _agg_call(z2, src, dst, zero_acc)
    z3 = _mid_call(p2, z2, dinv, b2.reshape(1, _D), W3)
    p3 = _agg_call(z3, src, dst, zero_acc)
    return _fin_call(p3, z3, dinv, b3.reshape(1, _D))
